# baseline (speedup 1.0000x reference)
import functools

import jax
import jax.numpy as jnp
from jax import lax
from jax.experimental import pallas as pl
from jax.experimental.pallas import tpu as pltpu
from jax.experimental.pallas import tpu_sc as plsc

B = 1024
D = 32
M = 100000
K = 16
BM = 2048
SUB = 256
NBLK = 49
MP = NBLK * BM
GW = 32
NG = MP // GW
NSEL = 18
NCAND = NSEL * GW
NEG = float("-inf")
BIGI = 2**31 - 1

NC, NS = 2, 16
NW = NC * NS


def _pcall(*args, **kw):
    return pl.pallas_call(*args, **kw)


def _k1_body(q_ref, p_ref, g_ref):
    i = pl.program_id(0)
    q = q_ref[...]
    for a in range(BM // SUB):
        st = lax.dot_general(p_ref[a * SUB:(a + 1) * SUB, :], q,
                             (((1,), (1,)), ((), ())),
                             preferred_element_type=jnp.float32)
        rowi = lax.broadcasted_iota(jnp.int32, (SUB, B), 0) + i * BM + a * SUB
        st = jnp.where(rowi < M, st, NEG)
        for t in range(SUB // GW):
            g = a * (SUB // GW) + t
            m = jnp.max(st[t * GW:(t + 1) * GW, :], axis=0, keepdims=True)
            g_ref[g:g + 1, :] = m


def _groupmax(q, patterns):
    return _pcall(
        _k1_body,
        grid=(NBLK,),
        in_specs=[
            pl.BlockSpec((B, D), lambda i: (0, 0)),
            pl.BlockSpec((BM, D), lambda i: (i, 0)),
        ],
        out_specs=pl.BlockSpec((BM // GW, B), lambda i: (i, 0)),
        out_shape=jax.ShapeDtypeStruct((NG, B), jnp.float32),
    )(q, patterns)


def _k2_body(g_ref, grp_ref):
    lb = g_ref.shape[1]
    gv = g_ref[...]
    rowi = lax.broadcasted_iota(jnp.int32, (NG, lb), 0)
    for k in range(NSEL):
        m = jnp.max(gv, axis=0, keepdims=True)
        idx = jnp.min(jnp.where(gv == m, rowi, BIGI), axis=0, keepdims=True)
        grp_ref[k:k + 1, :] = idx
        gv = jnp.where(rowi == idx, NEG, gv)


def _top_groups(gmax):
    lblk = 256
    return _pcall(
        _k2_body,
        grid=(B // lblk,),
        in_specs=[pl.BlockSpec((NG, lblk), lambda i: (0, i))],
        out_specs=pl.BlockSpec((NSEL, lblk), lambda i: (0, i)),
        out_shape=jax.ShapeDtypeStruct((NSEL, B), jnp.int32),
    )(gmax)


def _k2b_body(grp_ref, o_ref):
    lane = lax.broadcasted_iota(jnp.int32, (B, GW), 1)
    for j in range(NSEL):
        o_ref[:, j * GW:(j + 1) * GW] = grp_ref[:, j:j + 1] * GW + lane


def _expand_idx(grp):
    return _pcall(
        _k2b_body,
        out_shape=jax.ShapeDtypeStruct((B, NCAND), jnp.int32),
    )(grp)


def _sc_gather_rows(table, idx, width):
    n = idx.shape[0]
    per_w = n // NW
    chunk = next(c for c in range(min(128, per_w), 0, -1) if per_w % c == 0)
    nch = per_w // chunk
    nb = next(c for c in range(min(8, nch), 0, -1) if nch % c == 0)
    nbatch = nch // nb
    mesh = plsc.VectorSubcoreMesh(core_axis_name="c", subcore_axis_name="s",
                                  num_cores=NC, num_subcores=NS)

    @functools.partial(
        pl.kernel,
        out_type=jax.ShapeDtypeStruct((n, width), jnp.float32),
        mesh=mesh,
        compiler_params=pltpu.CompilerParams(use_tc_tiling_on_sc=False),
        scratch_types=[
            pltpu.VMEM((per_w,), jnp.int32),
            pltpu.VMEM((nb * chunk, width), jnp.float32),
            pltpu.SemaphoreType.DMA,
            pltpu.SemaphoreType.DMA,
        ],
    )
    def gather(table_hbm, idx_hbm, out_hbm, idx_v, rows_v, gsem, osem):
        wid = lax.axis_index("s") * NC + lax.axis_index("c")
        base = wid * per_w
        pltpu.sync_copy(idx_hbm.at[pl.ds(base, per_w)], idx_v)

        def batch(bi, _):
            gs = [
                pltpu.async_copy(
                    table_hbm.at[idx_v.at[pl.ds((bi * nb + u) * chunk, chunk)]],
                    rows_v.at[pl.ds(u * chunk, chunk)],
                    gsem,
                )
                for u in range(nb)
            ]
            for h in gs:
                h.wait()
            os = [
                pltpu.async_copy(
                    rows_v.at[pl.ds(u * chunk, chunk)],
                    out_hbm.at[pl.ds(base + (bi * nb + u) * chunk, chunk)],
                    osem,
                )
                for u in range(nb)
            ]
            for h in os:
                h.wait()
            return _

        lax.fori_loop(0, nbatch, batch, None)

    return gather(table, idx)


def _k5_body(cp_ref, pidx_ref, q_ref, o_ref):
    bb = q_ref.shape[0]
    cp = cp_ref[...]
    qv = q_ref[...]
    sc = jnp.sum(cp * qv[:, None, :], axis=2)
    gcol = pidx_ref[...]
    tops, tidx = [], []
    for k in range(K):
        m = jnp.max(sc, axis=1, keepdims=True)
        ti = jnp.min(jnp.where(sc == m, gcol, BIGI), axis=1, keepdims=True)
        tops.append(m)
        tidx.append(ti)
        sc = jnp.where(gcol == ti, NEG, sc)
    ts = jnp.concatenate(tops, axis=1)
    e = jnp.exp(ts - ts[:, 0:1])
    w = e / jnp.sum(e, axis=1, keepdims=True)
    wv = jnp.zeros((bb, NCAND), jnp.float32)
    for k in range(K):
        wv = jnp.where(gcol == tidx[k], w[:, k:k + 1], wv)
    o_ref[...] = jnp.sum(cp * wv[:, :, None], axis=1)


def _score_topk_retrieve(cp3, pidx, q):
    bblk = 32
    return _pcall(
        _k5_body,
        grid=(B // bblk,),
        in_specs=[
            pl.BlockSpec((bblk, NCAND, D), lambda i: (i, 0, 0)),
            pl.BlockSpec((bblk, NCAND), lambda i: (i, 0)),
            pl.BlockSpec((bblk, D), lambda i: (i, 0)),
        ],
        out_specs=pl.BlockSpec((bblk, D), lambda i: (i, 0)),
        out_shape=jax.ShapeDtypeStruct((B, D), jnp.float32),
    )(cp3, pidx, q)


def kernel(query, patterns, top_k):
    del top_k
    gmax = _groupmax(query, patterns)
    grp = _top_groups(gmax)
    pidx = _expand_idx(jnp.transpose(grp))
    cp = _sc_gather_rows(patterns, pidx.reshape(-1), D)
    return _score_topk_retrieve(cp.reshape(B, NCAND, D), pidx, query)
